# Initial kernel scaffold; baseline (speedup 1.0000x reference)
import jax
import jax.numpy as jnp
from jax.experimental import pallas as pl


def kernel(x, edge_index, W1, b1, W2, b2, W3, b3, Wc, bc):
    raise NotImplementedError("write your pallas kernel here")


# speedup vs baseline: 5.5888x; 5.5888x over previous
import functools

import jax
import jax.numpy as jnp
from jax import lax
from jax.experimental import pallas as pl
from jax.experimental.pallas import tpu as pltpu
from jax.experimental.pallas import tpu_sc as plsc

N = 10000
NPAD = 10112
E = 320000
CHUNK = 128
NTILES = 16
NSC = 2
NW = NSC * NTILES
RPT = 80
EPAD = NW * RPT * CHUNK
DEGROWS = 2 * NPAD
DEGW = 16

_mesh = plsc.VectorSubcoreMesh(core_axis_name="c", subcore_axis_name="s")
_sc_params = pltpu.CompilerParams(use_tc_tiling_on_sc=False)


def _deg_body(src_hbm, dst_hbm, zeros_hbm, ones_hbm, out_hbm,
              src_v, dst_v, ones_v, deg_sh):
    c = lax.axis_index("c")
    s = lax.axis_index("s")
    wid = c * NTILES + s
    ri = DEGROWS // NTILES
    pltpu.sync_copy(zeros_hbm.at[pl.ds(s * ri, ri)],
                    deg_sh.at[pl.ds(s * ri, ri)])
    pltpu.sync_copy(ones_hbm, ones_v)
    base = wid * RPT
    pltpu.sync_copy(src_hbm.at[pl.ds(base, RPT)], src_v)
    pltpu.sync_copy(dst_hbm.at[pl.ds(base, RPT)], dst_v)
    plsc.subcore_barrier()

    def body(j, carry):
        pltpu.sync_copy(ones_v, deg_sh.at[src_v.at[j]], add=True)
        pltpu.sync_copy(ones_v, deg_sh.at[dst_v.at[j]], add=True)
        return carry

    lax.fori_loop(0, RPT, body, 0)
    plsc.subcore_barrier()
    pltpu.sync_copy(deg_sh.at[pl.ds(s * ri, ri)],
                    out_hbm.at[c, pl.ds(s * ri, ri)])


def _make_deg_kernel():
    return pl.kernel(
        _deg_body,
        out_type=jax.ShapeDtypeStruct((NSC, DEGROWS, DEGW), jnp.float32),
        mesh=_mesh,
        compiler_params=_sc_params,
        scratch_types=[
            pltpu.VMEM((RPT, CHUNK), jnp.int32),
            pltpu.VMEM((RPT, CHUNK), jnp.int32),
            pltpu.VMEM((CHUNK, DEGW), jnp.float32),
            pltpu.VMEM_SHARED((DEGROWS, DEGW), jnp.float32),
        ],
    )


def _agg_body(d, src_hbm, dst_hbm, h_hbm, zeros_hbm, out_hbm,
              src_v, dst_v, rows_v, agg_sh, sem):
    c = lax.axis_index("c")
    s = lax.axis_index("s")
    wid = c * NTILES + s
    ri = NPAD // NTILES
    pltpu.sync_copy(zeros_hbm.at[pl.ds(s * ri, ri)],
                    agg_sh.at[pl.ds(s * ri, ri)])
    base = wid * RPT
    pltpu.sync_copy(src_hbm.at[pl.ds(base, RPT)], src_v)
    pltpu.sync_copy(dst_hbm.at[pl.ds(base, RPT)], dst_v)
    plsc.subcore_barrier()

    def body(j, carry):
        pltpu.async_copy(h_hbm.at[src_v.at[j]], rows_v, sem).wait()
        pltpu.sync_copy(rows_v, agg_sh.at[dst_v.at[j]], add=True)
        return carry

    lax.fori_loop(0, RPT, body, 0)
    plsc.subcore_barrier()
    pltpu.sync_copy(agg_sh.at[pl.ds(s * ri, ri)],
                    out_hbm.at[c, pl.ds(s * ri, ri)])


def _make_agg_kernel(d):
    return pl.kernel(
        functools.partial(_agg_body, d),
        out_type=jax.ShapeDtypeStruct((NSC, NPAD, d), jnp.float32),
        mesh=_mesh,
        compiler_params=_sc_params,
        scratch_types=[
            pltpu.VMEM((RPT, CHUNK), jnp.int32),
            pltpu.VMEM((RPT, CHUNK), jnp.int32),
            pltpu.VMEM((CHUNK, d), jnp.float32),
            pltpu.VMEM_SHARED((NPAD, d), jnp.float32),
            pltpu.SemaphoreType.DMA,
        ],
    )


def _tc1_body(deg_ref, x_ref, w1_ref, h_ref, sin_ref, sout_ref):
    dsum = deg_ref[0] + deg_ref[1]
    dout = dsum[0:N, 0:1]
    din = dsum[NPAD:NPAD + N, 0:1]
    s_out = lax.rsqrt(jnp.maximum(dout, 1.0))
    s_in = lax.rsqrt(jnp.maximum(din, 1.0))
    h = x_ref[...] * s_out
    h_ref[...] = jnp.dot(h, w1_ref[...], preferred_element_type=jnp.float32)
    sin_ref[...] = s_in
    sout_ref[...] = s_out


def _tc_mid_body(p_ref, sin_ref, sout_ref, b_ref, w_ref, out_ref):
    agg = p_ref[0, 0:N] + p_ref[1, 0:N]
    h = jnp.maximum(agg * sin_ref[...] + b_ref[...], 0.0)
    out_ref[...] = jnp.dot(h * sout_ref[...], w_ref[...],
                           preferred_element_type=jnp.float32)


def _tc_final_body(p_ref, sin_ref, b_ref, wc_ref, bc_ref, out_ref):
    agg = p_ref[0, 0:N] + p_ref[1, 0:N]
    h = jnp.maximum(agg * sin_ref[...] + b_ref[...], 0.0)
    hg = jnp.mean(h, axis=0, keepdims=True)
    out_ref[...] = jnp.dot(hg, wc_ref[...],
                           preferred_element_type=jnp.float32) + bc_ref[...]


def kernel(x, edge_index, W1, b1, W2, b2, W3, b3, Wc, bc):
    src = edge_index[0]
    dst = edge_index[1]
    pad = EPAD - E
    src_g = jnp.concatenate([src, jnp.zeros((pad,), jnp.int32)]).reshape(-1, CHUNK)
    dst_s = jnp.concatenate([dst, jnp.full((pad,), N, jnp.int32)]).reshape(-1, CHUNK)
    src_d = jnp.concatenate([src, jnp.full((pad,), N, jnp.int32)]).reshape(-1, CHUNK)
    dst_d = dst_s + NPAD

    zdeg = jnp.zeros((DEGROWS, DEGW), jnp.float32)
    ones = jnp.ones((CHUNK, DEGW), jnp.float32)

    degp = _make_deg_kernel()(src_d, dst_d, zdeg, ones)

    h1, s_in, s_out = pl.pallas_call(
        _tc1_body,
        out_shape=(
            jax.ShapeDtypeStruct((N, 64), jnp.float32),
            jax.ShapeDtypeStruct((N, 1), jnp.float32),
            jax.ShapeDtypeStruct((N, 1), jnp.float32),
        ),
    )(degp, x, W1)

    p1 = _make_agg_kernel(64)(src_g, dst_s, h1, jnp.zeros((NPAD, 64), jnp.float32))
    h2 = pl.pallas_call(
        _tc_mid_body,
        out_shape=jax.ShapeDtypeStruct((N, 32), jnp.float32),
    )(p1, s_in, s_out, b1, W2)

    p2 = _make_agg_kernel(32)(src_g, dst_s, h2, jnp.zeros((NPAD, 32), jnp.float32))
    h3 = pl.pallas_call(
        _tc_mid_body,
        out_shape=jax.ShapeDtypeStruct((N, 16), jnp.float32),
    )(p2, s_in, s_out, b2, W3)

    p3 = _make_agg_kernel(16)(src_g, dst_s, h3, jnp.zeros((NPAD, 16), jnp.float32))
    out = pl.pallas_call(
        _tc_final_body,
        out_shape=jax.ShapeDtypeStruct((1, 1), jnp.float32),
    )(p3, s_in, b3, Wc, bc)

    return out.reshape((1,))
